# baseline (speedup 1.0000x reference)
import functools

import jax
import jax.numpy as jnp
from jax import lax
from jax.experimental import pallas as pl
from jax.experimental.pallas import tpu as pltpu
from jax.experimental.pallas import tpu_sc as plsc

NUM_EMB = 1000000
DIM = 64
BATCH = 16384
NUM_WORKERS = 32
B_PER_W = BATCH // NUM_WORKERS
NGRP = B_PER_W // 16
MAX_T = NUM_EMB // 128 - 1
TAIL_LO = NUM_EMB - DIM
TAIL_START = NUM_EMB - 128


def kernel(x, weight):
    wt = weight.T
    tail = lax.slice(wt, (0, TAIL_START), (DIM, NUM_EMB))
    mesh = plsc.VectorSubcoreMesh(core_axis_name="c", subcore_axis_name="s")

    @functools.partial(
        pl.kernel,
        mesh=mesh,
        out_type=jax.ShapeDtypeStruct((DIM, BATCH), jnp.float32),
        scratch_types=[
            pltpu.VMEM((B_PER_W,), jnp.int32),
            *[pltpu.VMEM((DIM, 128), jnp.float32) for _ in range(8)],
            pltpu.VMEM((DIM, 128), jnp.float32),
            pltpu.VMEM((DIM, B_PER_W), jnp.float32),
            *[pltpu.SemaphoreType.DMA for _ in range(8)],
        ],
        compiler_params=pltpu.CompilerParams(needs_layout_passes=False),
    )
    def body(x_hbm, w_hbm, tail_hbm, out_hbm, xi_v, *rest):
        slabs = rest[0:8]
        tail_v = rest[8]
        stage = rest[9]
        sems = rest[10:18]
        wid = lax.axis_index("s") * 2 + lax.axis_index("c")
        base = wid * B_PER_W
        pltpu.sync_copy(x_hbm.at[pl.ds(base, B_PER_W)], xi_v)
        pltpu.sync_copy(tail_hbm, tail_v)

        dvecs = [lax.iota(jnp.int32, 16) + 16 * q for q in range(4)]

        def window_start(i):
            t = jnp.minimum(lax.shift_right_logical(i, 7), jnp.int32(MAX_T))
            return pl.multiple_of(t * 128, 128)

        def fire(i, b):
            start = window_start(i)
            for gg in range(8):
                pltpu.async_copy(
                    w_hbm.at[pl.ds(gg * 8, 8), pl.ds(start, 128)],
                    slabs[b].at[pl.ds(gg * 8, 8), :],
                    sems[b],
                )

        first = xi_v[pl.ds(0, 16)]
        for b in range(8):
            fire(first[b], b)

        @pl.loop(0, NGRP)
        def grp_loop(g):
            cur = xi_v[pl.ds(g * 16, 16)]
            nxt = xi_v[pl.ds(jnp.minimum(g + 1, NGRP - 1) * 16, 16)]
            for lane in range(16):
                k = g * 16 + lane
                b = lane % 8
                pltpu.make_async_copy(
                    w_hbm.at[:, pl.ds(0, 128)], slabs[b], sems[b]
                ).wait()
                i = cur[lane]
                col = jnp.minimum(i - window_start(i), jnp.int32(127))
                csplat = jnp.full((16,), col, jnp.int32)
                ksplat = jnp.full((16,), k, jnp.int32)
                for q in range(4):
                    v = plsc.load_gather(slabs[b], [dvecs[q], csplat])
                    plsc.store_scatter(stage, [dvecs[q], ksplat], v)
                nxt_i = cur[lane + 8] if lane < 8 else nxt[lane - 8]
                fire(nxt_i, b)

        for b in range(8):
            pltpu.make_async_copy(
                w_hbm.at[:, pl.ds(0, 128)], slabs[b], sems[b]
            ).wait()

        @pl.loop(0, NGRP)
        def tail_loop(g):
            cur = xi_v[pl.ds(g * 16, 16)]
            for lane in range(16):
                k = g * 16 + lane
                i = cur[lane]

                @pl.when(i >= TAIL_LO)
                def _():
                    col = i - TAIL_START
                    csplat = jnp.full((16,), col, jnp.int32)
                    ksplat = jnp.full((16,), k, jnp.int32)
                    for q in range(4):
                        v = plsc.load_gather(tail_v, [dvecs[q], csplat])
                        plsc.store_scatter(stage, [dvecs[q], ksplat], v)

        pltpu.sync_copy(stage, out_hbm.at[:, pl.ds(base, B_PER_W)])

    return body(x.astype(jnp.int32), wt, tail).T


# speedup vs baseline: 1.2466x; 1.2466x over previous
import functools

import jax
import jax.numpy as jnp
from jax import lax
from jax.experimental import pallas as pl
from jax.experimental.pallas import tpu as pltpu
from jax.experimental.pallas import tpu_sc as plsc

NUM_EMB = 1000000
DIM = 64
BATCH = 16384
NUM_WORKERS = 32
CW = 512
NCH0 = 61
SPAN = NCH0 * CW
TAIL_LO = NUM_EMB - DIM
TAIL_START = NUM_EMB - 128
CAP = 1024
SCAP = 2048
OUT_ROWS = BATCH + 256
BATCH_ROWS = 128


def kernel(x, weight):
    wt = weight.T
    tail = lax.slice(wt, (0, TAIL_START), (DIM, NUM_EMB))
    mesh = plsc.VectorSubcoreMesh(core_axis_name="c", subcore_axis_name="s")

    @functools.partial(
        pl.kernel,
        mesh=mesh,
        out_type=jax.ShapeDtypeStruct((OUT_ROWS, 128), jnp.float32),
        scratch_types=[
            pltpu.VMEM((4096,), jnp.int32),
            pltpu.VMEM((CAP,), jnp.int32),
            pltpu.VMEM((CAP,), jnp.int32),
            pltpu.VMEM((SCAP,), jnp.int32),
            pltpu.VMEM((SCAP,), jnp.int32),
            pltpu.VMEM((32,), jnp.int32),
            pltpu.VMEM((32,), jnp.int32),
            pltpu.VMEM((64,), jnp.int32),
            pltpu.VMEM((DIM, CW), jnp.float32),
            pltpu.VMEM((DIM, CW), jnp.float32),
            pltpu.VMEM((DIM, 128), jnp.float32),
            pltpu.VMEM((BATCH_ROWS, 128), jnp.float32),
            pltpu.VMEM((BATCH_ROWS,), jnp.int32),
            pltpu.SMEM((64,), jnp.int32),
            pltpu.SMEM((64,), jnp.int32),
            pltpu.SMEM((64,), jnp.int32),
            pltpu.SMEM((8,), jnp.int32),
            pltpu.SemaphoreType.DMA,
            pltpu.SemaphoreType.DMA,
            pltpu.SemaphoreType.DMA,
        ],
        compiler_params=pltpu.CompilerParams(needs_layout_passes=False),
    )
    def body(x_hbm, w_hbm, tail_hbm, out_hbm, xb_v, il_v, bl_v, si_v, sb_v,
             tl_v, tb_v, hist_v, cb0, cb1, tail_v, rows_v, blist_v,
             base_s, fill_s, cnt_s, m_s, sem0, sem1, fsem):
        wid = lax.axis_index("s") * 2 + lax.axis_index("c")
        lo = wid * SPAN
        nch = NCH0 + jnp.where(wid == NUM_WORKERS - 1, 1, 0).astype(jnp.int32)
        hi = lo + nch * CW

        cbs = (cb0, cb1)
        sems = (sem0, sem1)
        iota = lax.iota(jnp.int32, 16)
        lane0 = iota == 0
        ones = jnp.full((16,), 1, jnp.int32)
        dvecs = [iota + 16 * q for q in range(4)]

        pltpu.sync_copy(tail_hbm, tail_v)

        m_s[0] = jnp.int32(0)

        def refill_dumps():
            for j in range(BATCH_ROWS // 16):
                blist_v[pl.ds(j * 16, 16)] = BATCH + 16 * j + iota

        refill_dumps()

        losp = jnp.full((16,), lo, jnp.int32)
        hisp = jnp.full((16,), hi, jnp.int32)

        ptr0 = jnp.int32(0)
        tptr0 = jnp.int32(0)
        carry = (ptr0, tptr0)
        for piece in range(4):
            pltpu.sync_copy(x_hbm.at[pl.ds(piece * 4096, 4096)], xb_v)

            @pl.loop(0, 256, init_carry=carry)
            def match_loop(u, c):
                ptr, tptr = c
                xv = xb_v[pl.ds(u * 16, 16)]
                bvec = piece * 4096 + u * 16 + iota
                m = jnp.logical_and(xv >= losp, xv < hisp)
                cs = plsc.cumsum(jnp.where(m, 1, 0).astype(jnp.int32))
                pos = cs + (ptr - 1)
                plsc.store_scatter(il_v, [pos], xv, mask=m)
                plsc.store_scatter(bl_v, [pos], bvec, mask=m)
                mt = xv >= TAIL_LO
                cst = plsc.cumsum(jnp.where(mt, 1, 0).astype(jnp.int32))
                post = cst + (tptr - 1)
                plsc.store_scatter(tl_v, [post], xv, mask=mt)
                plsc.store_scatter(tb_v, [post], bvec, mask=mt)
                return (ptr + cs[15], tptr + cst[15])

            carry = match_loop
        nmatch, ntail = carry

        for j in range(4):
            hist_v[pl.ds(j * 16, 16)] = jnp.zeros((16,), jnp.int32)

        nmsp = jnp.full((16,), nmatch, jnp.int32)

        @pl.loop(0, CAP // 16)
        def hist_loop(u):
            iv = il_v[pl.ds(u * 16, 16)]
            valid = (u * 16 + iota) < nmsp
            cvec = lax.shift_right_arithmetic(iv - losp, 9)
            plsc.addupdate_scatter(hist_v, [cvec], ones, mask=valid)

        acc = jnp.int32(0)
        for j in range(4):
            h = hist_v[pl.ds(j * 16, 16)]
            ha = lax.shift_left(lax.shift_right_logical(h + 15, 4), 4)
            csa = plsc.cumsum(ha)
            starts = csa - ha + acc
            for lane in range(16):
                base_s[j * 16 + lane] = starts[lane]
                fill_s[j * 16 + lane] = starts[lane]
                cnt_s[j * 16 + lane] = h[lane]
            acc = acc + csa[15]

        @pl.loop(0, CAP // 16)
        def place_loop(u):
            iv = il_v[pl.ds(u * 16, 16)]
            bv = bl_v[pl.ds(u * 16, 16)]
            for lane in range(16):
                jj = u * 16 + lane

                @pl.when(jj < nmatch)
                def _():
                    i = iv[lane]
                    b = bv[lane]
                    cc = lax.shift_right_logical(i - lo, 9)
                    pos = fill_s[cc]
                    fill_s[cc] = pos + 1
                    psp = jnp.full((16,), pos, jnp.int32)
                    plsc.store_scatter(si_v, [psp], jnp.full((16,), i, jnp.int32), mask=lane0)
                    plsc.store_scatter(sb_v, [psp], jnp.full((16,), b, jnp.int32), mask=lane0)

        def flush():
            pltpu.async_copy(rows_v, out_hbm.at[blist_v], fsem).wait()
            refill_dumps()
            m_s[0] = jnp.int32(0)

        def emit(i, b, src_ref, col):
            mrow = m_s[0]
            csp = jnp.full((16,), col, jnp.int32)
            for q in range(4):
                v = plsc.load_gather(src_ref, [dvecs[q], csp])
                rows_v[mrow, pl.ds(q * 16, 16)] = v
            plsc.store_scatter(
                blist_v, [jnp.full((16,), mrow, jnp.int32)],
                jnp.full((16,), b, jnp.int32), mask=lane0)
            m_s[0] = mrow + 1

            @pl.when(mrow + 1 == BATCH_ROWS)
            def _():
                flush()

        def fire(cidx, bb):
            start = pl.multiple_of(lo + cidx * CW, 128)
            pltpu.async_copy(
                w_hbm.at[:, pl.ds(start, CW)], cbs[bb], sems[bb]
            )

        fire(jnp.int32(0), 0)
        fire(jnp.int32(1), 1)

        @pl.loop(0, (NCH0 + 2) // 2)
        def sweep_loop(g):
            for bb in range(2):
                cidx = g * 2 + bb

                @pl.when(cidx < nch)
                def _():
                    pltpu.make_async_copy(
                        w_hbm.at[:, pl.ds(0, CW)], cbs[bb], sems[bb]
                    ).wait()
                    chunk_lo = lo + cidx * CW
                    cbase = base_s[cidx]
                    cn = cnt_s[cidx]
                    ng = lax.shift_right_logical(cn + 15, 4)

                    @pl.loop(0, ng)
                    def grp(u):
                        iv = si_v[pl.ds(cbase + u * 16, 16)]
                        bv = sb_v[pl.ds(cbase + u * 16, 16)]
                        for lane in range(16):
                            jj = u * 16 + lane

                            @pl.when(jj < cn)
                            def _():
                                emit(iv[lane], bv[lane], cbs[bb],
                                     iv[lane] - chunk_lo)

                    fire(jnp.minimum(cidx + 2, nch - 1), bb)

        for bb in range(2):
            pltpu.make_async_copy(
                w_hbm.at[:, pl.ds(0, CW)], cbs[bb], sems[bb]
            ).wait()

        @pl.loop(0, 2)
        def tail_grp(u):
            iv = tl_v[pl.ds(u * 16, 16)]
            bv = tb_v[pl.ds(u * 16, 16)]
            for lane in range(16):
                jj = u * 16 + lane

                @pl.when(jj < ntail)
                def _():
                    emit(iv[lane], bv[lane], tail_v, iv[lane] - TAIL_START)

        @pl.when(m_s[0] > 0)
        def _():
            flush()

    out = body(x.astype(jnp.int32), wt, tail)
    return lax.slice(out, (0, 0), (BATCH, DIM))


# speedup vs baseline: 1.2844x; 1.0303x over previous
import functools

import jax
import jax.numpy as jnp
from jax import lax
from jax.experimental import pallas as pl
from jax.experimental.pallas import tpu as pltpu
from jax.experimental.pallas import tpu_sc as plsc

NUM_EMB = 1000000
DIM = 64
BATCH = 16384
NUM_WORKERS = 32
CW = 512
NCH0 = 61
SPAN = NCH0 * CW
TAIL_LO = NUM_EMB - DIM
TAIL_START = NUM_EMB - 128
CAP = 1024
SCAP = 2048
OUT_ROWS = BATCH + 256
BATCH_ROWS = 128


def kernel(x, weight):
    wt = weight.T
    tail = lax.slice(wt, (0, TAIL_START), (DIM, NUM_EMB))
    mesh = plsc.VectorSubcoreMesh(core_axis_name="c", subcore_axis_name="s")

    @functools.partial(
        pl.kernel,
        mesh=mesh,
        out_type=jax.ShapeDtypeStruct((OUT_ROWS, 128), jnp.float32),
        scratch_types=[
            pltpu.VMEM((4096,), jnp.int32),
            pltpu.VMEM((CAP,), jnp.int32),
            pltpu.VMEM((CAP,), jnp.int32),
            pltpu.VMEM((SCAP,), jnp.int32),
            pltpu.VMEM((SCAP,), jnp.int32),
            pltpu.VMEM((64,), jnp.int32),
            pltpu.VMEM((DIM, CW), jnp.float32),
            pltpu.VMEM((DIM, CW), jnp.float32),
            pltpu.VMEM((DIM, 128), jnp.float32),
            pltpu.VMEM((BATCH_ROWS, 128), jnp.float32),
            pltpu.VMEM((BATCH_ROWS,), jnp.int32),
            pltpu.SMEM((64,), jnp.int32),
            pltpu.SMEM((64,), jnp.int32),
            pltpu.SMEM((64,), jnp.int32),
            pltpu.SMEM((8,), jnp.int32),
            pltpu.SemaphoreType.DMA,
            pltpu.SemaphoreType.DMA,
            pltpu.SemaphoreType.DMA,
        ],
        compiler_params=pltpu.CompilerParams(needs_layout_passes=False),
    )
    def body(x_hbm, w_hbm, tail_hbm, out_hbm, xb_v, il_v, bl_v, si_v, sb_v,
             hist_v, cb0, cb1, tail_v, rows_v, blist_v,
             base_s, fill_s, cnt_s, m_s, sem0, sem1, fsem):
        wid = lax.axis_index("s") * 2 + lax.axis_index("c")
        lo = wid * SPAN
        is_last = wid == NUM_WORKERS - 1
        nch = NCH0 + jnp.where(is_last, 1, 0).astype(jnp.int32)
        hi = lo + nch * CW + jnp.where(is_last, DIM, 0).astype(jnp.int32)

        cbs = (cb0, cb1)
        sems = (sem0, sem1)
        iota = lax.iota(jnp.int32, 16)
        lane0 = iota == 0
        ones = jnp.full((16,), 1, jnp.int32)
        dvecs = [iota + 16 * q for q in range(4)]

        pltpu.sync_copy(tail_hbm, tail_v)

        def fire(cidx, bb):
            start = pl.multiple_of(lo + cidx * CW, 128)
            pltpu.async_copy(
                w_hbm.at[:, pl.ds(start, CW)], cbs[bb], sems[bb]
            )

        fire(jnp.int32(0), 0)
        fire(jnp.int32(1), 1)

        m_s[0] = jnp.int32(0)

        def refill_dumps():
            for j in range(BATCH_ROWS // 16):
                blist_v[pl.ds(j * 16, 16)] = BATCH + 16 * j + iota

        refill_dumps()

        losp = jnp.full((16,), lo, jnp.int32)
        hisp = jnp.full((16,), hi, jnp.int32)

        carry = jnp.int32(0)
        for piece in range(4):
            pltpu.sync_copy(x_hbm.at[pl.ds(piece * 4096, 4096)], xb_v)

            @pl.loop(0, 256, init_carry=carry)
            def match_loop(u, ptr):
                xv = xb_v[pl.ds(u * 16, 16)]
                bvec = piece * 4096 + u * 16 + iota
                m = jnp.logical_and(xv >= losp, xv < hisp)
                cs = plsc.cumsum(jnp.where(m, 1, 0).astype(jnp.int32))
                pos = cs + (ptr - 1)
                plsc.store_scatter(il_v, [pos], xv, mask=m)
                plsc.store_scatter(bl_v, [pos], bvec, mask=m)
                return ptr + cs[15]

            carry = match_loop
        nmatch = carry

        for j in range(4):
            hist_v[pl.ds(j * 16, 16)] = jnp.zeros((16,), jnp.int32)

        nmsp = jnp.full((16,), nmatch, jnp.int32)

        ngm = lax.shift_right_logical(nmatch + 15, 4)

        @pl.loop(0, ngm)
        def hist_loop(u):
            iv = il_v[pl.ds(u * 16, 16)]
            valid = (u * 16 + iota) < nmsp
            cvec = lax.shift_right_arithmetic(iv - losp, 9)
            plsc.addupdate_scatter(hist_v, [cvec], ones, mask=valid)

        acc = jnp.int32(0)
        for j in range(4):
            h = hist_v[pl.ds(j * 16, 16)]
            ha = lax.shift_left(lax.shift_right_logical(h + 15, 4), 4)
            csa = plsc.cumsum(ha)
            starts = csa - ha + acc
            for lane in range(16):
                base_s[j * 16 + lane] = starts[lane]
                fill_s[j * 16 + lane] = starts[lane]
                cnt_s[j * 16 + lane] = h[lane]
            acc = acc + csa[15]

        @pl.loop(0, ngm)
        def place_loop(u):
            iv = il_v[pl.ds(u * 16, 16)]
            bv = bl_v[pl.ds(u * 16, 16)]
            for lane in range(16):
                jj = u * 16 + lane

                @pl.when(jj < nmatch)
                def _():
                    i = iv[lane]
                    b = bv[lane]
                    cc = lax.shift_right_logical(i - lo, 9)
                    pos = fill_s[cc]
                    fill_s[cc] = pos + 1
                    psp = jnp.full((16,), pos, jnp.int32)
                    plsc.store_scatter(si_v, [psp], jnp.full((16,), i, jnp.int32), mask=lane0)
                    plsc.store_scatter(sb_v, [psp], jnp.full((16,), b, jnp.int32), mask=lane0)

        def flush():
            pltpu.async_copy(rows_v, out_hbm.at[blist_v], fsem).wait()
            refill_dumps()
            m_s[0] = jnp.int32(0)

        def emit(i, b, src_ref, col):
            mrow = m_s[0]
            csp = jnp.full((16,), col, jnp.int32)
            for q in range(4):
                v = plsc.load_gather(src_ref, [dvecs[q], csp])
                rows_v[mrow, pl.ds(q * 16, 16)] = v
            plsc.store_scatter(
                blist_v, [jnp.full((16,), mrow, jnp.int32)],
                jnp.full((16,), b, jnp.int32), mask=lane0)
            m_s[0] = mrow + 1

            @pl.when(mrow + 1 == BATCH_ROWS)
            def _():
                flush()

        @pl.loop(0, (NCH0 + 2) // 2)
        def sweep_loop(g):
            for bb in range(2):
                cidx = g * 2 + bb

                @pl.when(cidx < nch)
                def _():
                    pltpu.make_async_copy(
                        w_hbm.at[:, pl.ds(0, CW)], cbs[bb], sems[bb]
                    ).wait()
                    chunk_lo = lo + cidx * CW
                    cbase = base_s[cidx]
                    cn = cnt_s[cidx]
                    ng = lax.shift_right_logical(cn + 15, 4)

                    @pl.loop(0, ng)
                    def grp(u):
                        iv = si_v[pl.ds(cbase + u * 16, 16)]
                        bv = sb_v[pl.ds(cbase + u * 16, 16)]
                        for lane in range(16):
                            jj = u * 16 + lane

                            @pl.when(jj < cn)
                            def _():
                                emit(iv[lane], bv[lane], cbs[bb],
                                     iv[lane] - chunk_lo)

                    fire(jnp.minimum(cidx + 2, nch - 1), bb)

        for bb in range(2):
            pltpu.make_async_copy(
                w_hbm.at[:, pl.ds(0, CW)], cbs[bb], sems[bb]
            ).wait()

        tbase = base_s[62]
        tn = cnt_s[62]
        ngt = lax.shift_right_logical(tn + 15, 4)

        @pl.loop(0, ngt)
        def tail_grp(u):
            iv = si_v[pl.ds(tbase + u * 16, 16)]
            bv = sb_v[pl.ds(tbase + u * 16, 16)]
            for lane in range(16):
                jj = u * 16 + lane

                @pl.when(jj < tn)
                def _():
                    emit(iv[lane], bv[lane], tail_v, iv[lane] - TAIL_START)

        @pl.when(m_s[0] > 0)
        def _():
            flush()

    out = body(x.astype(jnp.int32), wt, tail)
    return lax.slice(out, (0, 0), (BATCH, DIM))


# speedup vs baseline: 1.4218x; 1.1070x over previous
import functools

import jax
import jax.numpy as jnp
from jax import lax
from jax.experimental import pallas as pl
from jax.experimental.pallas import tpu as pltpu
from jax.experimental.pallas import tpu_sc as plsc

NUM_EMB = 1000000
DIM = 64
BATCH = 16384
NUM_WORKERS = 32
CW = 512
NCH0 = 61
SPAN = NCH0 * CW
TAIL_LO = NUM_EMB - DIM
TAIL_START = NUM_EMB - 128
CAP = 896
SCAP = 1832
OUT_ROWS = BATCH + 256
BATCH_ROWS = 128


def kernel(x, weight):
    wt = weight.T
    tail = lax.slice(wt, (0, TAIL_START), (DIM, NUM_EMB))
    mesh = plsc.VectorSubcoreMesh(core_axis_name="c", subcore_axis_name="s")

    @functools.partial(
        pl.kernel,
        mesh=mesh,
        out_type=jax.ShapeDtypeStruct((OUT_ROWS, 128), jnp.float32),
        scratch_types=[
            pltpu.VMEM((2048,), jnp.int32),
            pltpu.VMEM((CAP,), jnp.int32),
            pltpu.VMEM((CAP,), jnp.int32),
            pltpu.VMEM((SCAP,), jnp.int32),
            pltpu.VMEM((SCAP,), jnp.int32),
            pltpu.VMEM((64,), jnp.int32),
            pltpu.VMEM((DIM, CW), jnp.float32),
            pltpu.VMEM((DIM, CW), jnp.float32),
            pltpu.VMEM((DIM, CW), jnp.float32),
            pltpu.VMEM((DIM, 128), jnp.float32),
            pltpu.VMEM((BATCH_ROWS, 128), jnp.float32),
            pltpu.VMEM((BATCH_ROWS,), jnp.int32),
            pltpu.SMEM((64,), jnp.int32),
            pltpu.SMEM((64,), jnp.int32),
            pltpu.SMEM((64,), jnp.int32),
            pltpu.SMEM((8,), jnp.int32),
            pltpu.SemaphoreType.DMA,
            pltpu.SemaphoreType.DMA,
            pltpu.SemaphoreType.DMA,
            pltpu.SemaphoreType.DMA,
        ],
        compiler_params=pltpu.CompilerParams(needs_layout_passes=False),
    )
    def body(x_hbm, w_hbm, tail_hbm, out_hbm, xb_v, il_v, bl_v, si_v, sb_v,
             hist_v, cb0, cb1, cb2, tail_v, rows_v, blist_v,
             base_s, fill_s, cnt_s, m_s, sem0, sem1, sem2, fsem):
        wid = lax.axis_index("s") * 2 + lax.axis_index("c")
        lo = wid * SPAN
        is_last = wid == NUM_WORKERS - 1
        nch = NCH0 + jnp.where(is_last, 1, 0).astype(jnp.int32)
        hi = lo + nch * CW + jnp.where(is_last, DIM, 0).astype(jnp.int32)

        cbs = (cb0, cb1, cb2)
        sems = (sem0, sem1, sem2)
        iota = lax.iota(jnp.int32, 16)
        lane0 = iota == 0
        ones = jnp.full((16,), 1, jnp.int32)
        dvecs = [iota + 16 * q for q in range(4)]

        pltpu.sync_copy(tail_hbm, tail_v)

        def fire(cidx, bb):
            start = pl.multiple_of(
                lo + jnp.minimum(cidx, nch - 1) * CW, 128)
            pltpu.async_copy(
                w_hbm.at[:, pl.ds(start, CW)], cbs[bb], sems[bb]
            )

        fire(jnp.int32(0), 0)
        fire(jnp.int32(1), 1)

        m_s[0] = jnp.int32(0)

        def refill_dumps():
            for j in range(BATCH_ROWS // 16):
                blist_v[pl.ds(j * 16, 16)] = BATCH + 16 * j + iota

        refill_dumps()

        losp = jnp.full((16,), lo, jnp.int32)
        hisp = jnp.full((16,), hi, jnp.int32)

        carry = jnp.int32(0)
        for piece in range(8):
            pltpu.sync_copy(x_hbm.at[pl.ds(piece * 2048, 2048)], xb_v)

            @pl.loop(0, 128, init_carry=carry)
            def match_loop(u, ptr):
                xv = xb_v[pl.ds(u * 16, 16)]
                bvec = piece * 2048 + u * 16 + iota
                m = jnp.logical_and(xv >= losp, xv < hisp)
                cs = plsc.cumsum(jnp.where(m, 1, 0).astype(jnp.int32))
                pos = cs + (ptr - 1)
                plsc.store_scatter(il_v, [pos], xv, mask=m)
                plsc.store_scatter(bl_v, [pos], bvec, mask=m)
                return ptr + cs[15]

            carry = match_loop
        nmatch = carry

        for j in range(4):
            hist_v[pl.ds(j * 16, 16)] = jnp.zeros((16,), jnp.int32)

        nmsp = jnp.full((16,), nmatch, jnp.int32)

        ngm = lax.shift_right_logical(nmatch + 15, 4)

        @pl.loop(0, ngm)
        def hist_loop(u):
            iv = il_v[pl.ds(u * 16, 16)]
            valid = (u * 16 + iota) < nmsp
            cvec = lax.shift_right_arithmetic(iv - losp, 9)
            plsc.addupdate_scatter(hist_v, [cvec], ones, mask=valid)

        acc = jnp.int32(0)
        for j in range(4):
            h = hist_v[pl.ds(j * 16, 16)]
            ha = lax.shift_left(lax.shift_right_logical(h + 15, 4), 4)
            csa = plsc.cumsum(ha)
            starts = csa - ha + acc
            for lane in range(16):
                base_s[j * 16 + lane] = starts[lane]
                fill_s[j * 16 + lane] = starts[lane]
                cnt_s[j * 16 + lane] = h[lane]
            acc = acc + csa[15]

        @pl.loop(0, ngm)
        def place_loop(u):
            iv = il_v[pl.ds(u * 16, 16)]
            bv = bl_v[pl.ds(u * 16, 16)]
            for lane in range(16):
                jj = u * 16 + lane

                @pl.when(jj < nmatch)
                def _():
                    i = iv[lane]
                    b = bv[lane]
                    cc = lax.shift_right_logical(i - lo, 9)
                    pos = fill_s[cc]
                    fill_s[cc] = pos + 1
                    psp = jnp.full((16,), pos, jnp.int32)
                    plsc.store_scatter(si_v, [psp], jnp.full((16,), i, jnp.int32), mask=lane0)
                    plsc.store_scatter(sb_v, [psp], jnp.full((16,), b, jnp.int32), mask=lane0)

        def flush():
            pltpu.async_copy(rows_v, out_hbm.at[blist_v], fsem).wait()
            refill_dumps()
            m_s[0] = jnp.int32(0)

        def emit(i, b, src_ref, col):
            mrow = m_s[0]
            csp = jnp.full((16,), col, jnp.int32)
            for q in range(4):
                v = plsc.load_gather(src_ref, [dvecs[q], csp])
                rows_v[mrow, pl.ds(q * 16, 16)] = v
            plsc.store_scatter(
                blist_v, [jnp.full((16,), mrow, jnp.int32)],
                jnp.full((16,), b, jnp.int32), mask=lane0)
            m_s[0] = mrow + 1

            @pl.when(mrow + 1 == BATCH_ROWS)
            def _():
                flush()

        NCHU = NCH0 + 1

        @pl.loop(0, (NCHU + 2) // 3)
        def sweep_loop(g):
            for bb in range(3):
                cidx = g * 3 + bb

                @pl.when(cidx < NCHU)
                def _():
                    pltpu.make_async_copy(
                        w_hbm.at[:, pl.ds(0, CW)], cbs[bb], sems[bb]
                    ).wait()
                    fire(cidx + 2, (bb + 2) % 3)
                    chunk_lo = lo + cidx * CW
                    cbase = base_s[cidx]
                    cn = cnt_s[cidx]
                    ng = lax.shift_right_logical(cn + 15, 4)

                    @pl.loop(0, ng)
                    def grp(u):
                        iv = si_v[pl.ds(cbase + u * 16, 16)]
                        bv = sb_v[pl.ds(cbase + u * 16, 16)]
                        for lane in range(16):
                            jj = u * 16 + lane

                            @pl.when(jj < cn)
                            def _():
                                emit(iv[lane], bv[lane], cbs[bb],
                                     iv[lane] - chunk_lo)

        for bb in (2, 0):
            pltpu.make_async_copy(
                w_hbm.at[:, pl.ds(0, CW)], cbs[bb], sems[bb]
            ).wait()

        tbase = base_s[62]
        tn = cnt_s[62]
        ngt = lax.shift_right_logical(tn + 15, 4)

        @pl.loop(0, ngt)
        def tail_grp(u):
            iv = si_v[pl.ds(tbase + u * 16, 16)]
            bv = sb_v[pl.ds(tbase + u * 16, 16)]
            for lane in range(16):
                jj = u * 16 + lane

                @pl.when(jj < tn)
                def _():
                    emit(iv[lane], bv[lane], tail_v, iv[lane] - TAIL_START)

        @pl.when(m_s[0] > 0)
        def _():
            flush()

    out = body(x.astype(jnp.int32), wt, tail)
    return lax.slice(out, (0, 0), (BATCH, DIM))


# speedup vs baseline: 1.4467x; 1.0175x over previous
import functools

import jax
import jax.numpy as jnp
from jax import lax
from jax.experimental import pallas as pl
from jax.experimental.pallas import tpu as pltpu
from jax.experimental.pallas import tpu_sc as plsc

NUM_EMB = 1000000
DIM = 64
BATCH = 16384
NUM_WORKERS = 32
CW = 512
NCH0 = 61
SPAN = NCH0 * CW
TAIL_LO = NUM_EMB - DIM
TAIL_START = NUM_EMB - 128
CAP = 896
SCAP = 1832
OUT_ROWS = BATCH + 256
BATCH_ROWS = 112


def kernel(x, weight):
    wt = weight.T
    tail = lax.slice(wt, (0, TAIL_START), (DIM, NUM_EMB))
    mesh = plsc.VectorSubcoreMesh(core_axis_name="c", subcore_axis_name="s")

    @functools.partial(
        pl.kernel,
        mesh=mesh,
        out_type=jax.ShapeDtypeStruct((OUT_ROWS, 128), jnp.float32),
        scratch_types=[
            pltpu.VMEM((2048,), jnp.int32),
            pltpu.VMEM((2048,), jnp.int32),
            pltpu.VMEM((CAP,), jnp.int32),
            pltpu.VMEM((CAP,), jnp.int32),
            pltpu.VMEM((SCAP,), jnp.int32),
            pltpu.VMEM((SCAP,), jnp.int32),
            pltpu.VMEM((64,), jnp.int32),
            pltpu.VMEM((DIM, CW), jnp.float32),
            pltpu.VMEM((DIM, CW), jnp.float32),
            pltpu.VMEM((DIM, CW), jnp.float32),
            pltpu.VMEM((DIM, 128), jnp.float32),
            pltpu.VMEM((BATCH_ROWS, 128), jnp.float32),
            pltpu.VMEM((BATCH_ROWS,), jnp.int32),
            pltpu.SMEM((64,), jnp.int32),
            pltpu.SMEM((64,), jnp.int32),
            pltpu.SMEM((64,), jnp.int32),
            pltpu.SMEM((8,), jnp.int32),
            pltpu.SemaphoreType.DMA,
            pltpu.SemaphoreType.DMA,
            pltpu.SemaphoreType.DMA,
            pltpu.SemaphoreType.DMA,
            pltpu.SemaphoreType.DMA,
            pltpu.SemaphoreType.DMA,
        ],
        compiler_params=pltpu.CompilerParams(needs_layout_passes=False),
    )
    def body(x_hbm, w_hbm, tail_hbm, out_hbm, xb0, xb1, il_v, bl_v,
             si_v, sb_v, hist_v, cb0, cb1, cb2, tail_v, rows_v, blist_v,
             base_s, fill_s, cnt_s, m_s, sem0, sem1, sem2, fsem, xsem0,
             xsem1):
        wid = lax.axis_index("s") * 2 + lax.axis_index("c")
        lo = wid * SPAN
        is_last = wid == NUM_WORKERS - 1
        nch = NCH0 + jnp.where(is_last, 1, 0).astype(jnp.int32)
        hi = lo + nch * CW + jnp.where(is_last, DIM, 0).astype(jnp.int32)

        cbs = (cb0, cb1, cb2)
        sems = (sem0, sem1, sem2)
        iota = lax.iota(jnp.int32, 16)
        lane0 = iota == 0
        ones = jnp.full((16,), 1, jnp.int32)
        dvecs = [iota + 16 * q for q in range(4)]

        tail_cp = pltpu.make_async_copy(tail_hbm, tail_v, fsem)
        tail_cp.start()

        def fire(cidx, bb):
            start = pl.multiple_of(
                lo + jnp.minimum(cidx, nch - 1) * CW, 128)
            pltpu.async_copy(
                w_hbm.at[:, pl.ds(start, CW)], cbs[bb], sems[bb]
            )

        fire(jnp.int32(0), 0)
        fire(jnp.int32(1), 1)

        m_s[0] = jnp.int32(0)

        def refill_dumps():
            for j in range(BATCH_ROWS // 16):
                blist_v[pl.ds(j * 16, 16)] = BATCH + 16 * j + iota

        refill_dumps()

        losp = jnp.full((16,), lo, jnp.int32)
        hisp = jnp.full((16,), hi, jnp.int32)

        carry = jnp.int32(0)
        xbs = (xb0, xb1)
        xsems = (xsem0, xsem1)

        def xfire(piece, xb):
            pltpu.async_copy(
                x_hbm.at[pl.ds(piece * 2048, 2048)], xbs[xb], xsems[xb])

        xfire(0, 0)
        xfire(1, 1)
        for piece in range(8):
            xb = piece % 2
            pltpu.make_async_copy(
                x_hbm.at[pl.ds(0, 2048)], xbs[xb], xsems[xb]).wait()
            if piece + 2 < 8:
                xfire(piece + 2, xb)
            xb_v = xbs[xb]

            @pl.loop(0, 128, init_carry=carry)
            def match_loop(u, ptr):
                xv = xb_v[pl.ds(u * 16, 16)]
                bvec = piece * 2048 + u * 16 + iota
                m = jnp.logical_and(xv >= losp, xv < hisp)
                cs = plsc.cumsum(jnp.where(m, 1, 0).astype(jnp.int32))
                pos = cs + (ptr - 1)
                plsc.store_scatter(il_v, [pos], xv, mask=m)
                plsc.store_scatter(bl_v, [pos], bvec, mask=m)
                return ptr + cs[15]

            carry = match_loop
        nmatch = carry

        for j in range(4):
            hist_v[pl.ds(j * 16, 16)] = jnp.zeros((16,), jnp.int32)

        nmsp = jnp.full((16,), nmatch, jnp.int32)

        ngm = lax.shift_right_logical(nmatch + 15, 4)

        @pl.loop(0, ngm)
        def hist_loop(u):
            iv = il_v[pl.ds(u * 16, 16)]
            valid = (u * 16 + iota) < nmsp
            cvec = lax.shift_right_arithmetic(iv - losp, 9)
            plsc.addupdate_scatter(hist_v, [cvec], ones, mask=valid)

        acc = jnp.int32(0)
        for j in range(4):
            h = hist_v[pl.ds(j * 16, 16)]
            ha = lax.shift_left(lax.shift_right_logical(h + 15, 4), 4)
            csa = plsc.cumsum(ha)
            starts = csa - ha + acc
            for lane in range(16):
                base_s[j * 16 + lane] = starts[lane]
                fill_s[j * 16 + lane] = starts[lane]
                cnt_s[j * 16 + lane] = h[lane]
            acc = acc + csa[15]

        @pl.loop(0, ngm)
        def place_loop(u):
            iv = il_v[pl.ds(u * 16, 16)]
            bv = bl_v[pl.ds(u * 16, 16)]
            for lane in range(16):
                jj = u * 16 + lane

                @pl.when(jj < nmatch)
                def _():
                    i = iv[lane]
                    b = bv[lane]
                    cc = lax.shift_right_logical(i - lo, 9)
                    pos = fill_s[cc]
                    fill_s[cc] = pos + 1
                    psp = jnp.full((16,), pos, jnp.int32)
                    plsc.store_scatter(si_v, [psp], jnp.full((16,), i, jnp.int32), mask=lane0)
                    plsc.store_scatter(sb_v, [psp], jnp.full((16,), b, jnp.int32), mask=lane0)

        def flush():
            pltpu.async_copy(rows_v, out_hbm.at[blist_v], fsem).wait()
            refill_dumps()
            m_s[0] = jnp.int32(0)

        def emit(i, b, src_ref, col):
            mrow = m_s[0]
            csp = jnp.full((16,), col, jnp.int32)
            for q in range(4):
                v = plsc.load_gather(src_ref, [dvecs[q], csp])
                rows_v[mrow, pl.ds(q * 16, 16)] = v
            plsc.store_scatter(
                blist_v, [jnp.full((16,), mrow, jnp.int32)],
                jnp.full((16,), b, jnp.int32), mask=lane0)
            m_s[0] = mrow + 1

            @pl.when(mrow + 1 == BATCH_ROWS)
            def _():
                flush()

        NCHU = NCH0 + 1

        @pl.loop(0, (NCHU + 2) // 3)
        def sweep_loop(g):
            for bb in range(3):
                cidx = g * 3 + bb

                @pl.when(cidx < NCHU)
                def _():
                    pltpu.make_async_copy(
                        w_hbm.at[:, pl.ds(0, CW)], cbs[bb], sems[bb]
                    ).wait()
                    fire(cidx + 2, (bb + 2) % 3)
                    chunk_lo = lo + cidx * CW
                    cbase = base_s[cidx]
                    cn = cnt_s[cidx]
                    ng = lax.shift_right_logical(cn + 15, 4)

                    @pl.loop(0, ng)
                    def grp(u):
                        iv = si_v[pl.ds(cbase + u * 16, 16)]
                        bv = sb_v[pl.ds(cbase + u * 16, 16)]
                        for lane in range(16):
                            jj = u * 16 + lane

                            @pl.when(jj < cn)
                            def _():
                                emit(iv[lane], bv[lane], cbs[bb],
                                     iv[lane] - chunk_lo)

        for bb in (2, 0):
            pltpu.make_async_copy(
                w_hbm.at[:, pl.ds(0, CW)], cbs[bb], sems[bb]
            ).wait()

        tail_cp.wait()
        tbase = base_s[62]
        tn = cnt_s[62]
        ngt = lax.shift_right_logical(tn + 15, 4)

        @pl.loop(0, ngt)
        def tail_grp(u):
            iv = si_v[pl.ds(tbase + u * 16, 16)]
            bv = sb_v[pl.ds(tbase + u * 16, 16)]
            for lane in range(16):
                jj = u * 16 + lane

                @pl.when(jj < tn)
                def _():
                    emit(iv[lane], bv[lane], tail_v, iv[lane] - TAIL_START)

        @pl.when(m_s[0] > 0)
        def _():
            flush()

    out = body(x.astype(jnp.int32), wt, tail)
    return lax.slice(out, (0, 0), (BATCH, DIM))


# speedup vs baseline: 1.4919x; 1.0313x over previous
import functools

import jax
import jax.numpy as jnp
from jax import lax
from jax.experimental import pallas as pl
from jax.experimental.pallas import tpu as pltpu
from jax.experimental.pallas import tpu_sc as plsc

NUM_EMB = 1000000
DIM = 64
BATCH = 16384
NUM_WORKERS = 32
CW = 512
NCH0 = 61
SPAN = NCH0 * CW
TAIL_LO = NUM_EMB - DIM
TAIL_START = NUM_EMB - 128
CAP = 896
SCAP = 1832
OUT_ROWS = BATCH + 256
BATCH_ROWS = 112


def kernel(x, weight):
    wt = weight.T
    tail = lax.slice(wt, (0, TAIL_START), (DIM, NUM_EMB))
    mesh = plsc.VectorSubcoreMesh(core_axis_name="c", subcore_axis_name="s")

    @functools.partial(
        pl.kernel,
        mesh=mesh,
        out_type=jax.ShapeDtypeStruct((OUT_ROWS, 128), jnp.float32),
        scratch_types=[
            pltpu.VMEM((2048,), jnp.int32),
            pltpu.VMEM((2048,), jnp.int32),
            pltpu.VMEM((CAP,), jnp.int32),
            pltpu.VMEM((CAP,), jnp.int32),
            pltpu.VMEM((SCAP,), jnp.int32),
            pltpu.VMEM((SCAP,), jnp.int32),
            pltpu.VMEM((64,), jnp.int32),
            pltpu.VMEM((DIM, CW), jnp.float32),
            pltpu.VMEM((DIM, CW), jnp.float32),
            pltpu.VMEM((DIM, CW), jnp.float32),
            pltpu.VMEM((DIM, 128), jnp.float32),
            pltpu.VMEM((BATCH_ROWS, 128), jnp.float32),
            pltpu.VMEM((BATCH_ROWS,), jnp.int32),
            pltpu.SMEM((64,), jnp.int32),
            pltpu.SMEM((64,), jnp.int32),
            pltpu.SMEM((64,), jnp.int32),
            pltpu.SMEM((8,), jnp.int32),
            *[pltpu.SemaphoreType.DMA for _ in range(7)],
        ],
        compiler_params=pltpu.CompilerParams(needs_layout_passes=False),
    )
    def body(x_hbm, w_hbm, tail_hbm, out_hbm, xb0, xb1, il_v, bl_v,
             si_v, sb_v, hist_v, cb0, cb1, cb2, tail_v, rows_v, blist_v,
             base_s, fill_s, cnt_s, m_s, sem0, sem1, sem2, fsem, xsem0,
             xsem1, tsem):
        wid = lax.axis_index("s") * 2 + lax.axis_index("c")
        lo = wid * SPAN
        is_last = wid == NUM_WORKERS - 1
        nch = NCH0 + jnp.where(is_last, 1, 0).astype(jnp.int32)
        hi = lo + nch * CW + jnp.where(is_last, DIM, 0).astype(jnp.int32)

        cbs = (cb0, cb1, cb2)
        sems = (sem0, sem1, sem2)
        iota = lax.iota(jnp.int32, 16)
        lane0 = iota == 0
        ones = jnp.full((16,), 1, jnp.int32)
        dvecs = [iota + 16 * q for q in range(4)]

        tail_cp = pltpu.make_async_copy(tail_hbm, tail_v, tsem)
        tail_cp.start()

        def fire(cidx, bb):
            start = pl.multiple_of(
                lo + jnp.minimum(cidx, nch - 1) * CW, 128)
            pltpu.async_copy(
                w_hbm.at[:, pl.ds(start, CW)], cbs[bb], sems[bb]
            )

        fire(jnp.int32(0), 0)
        fire(jnp.int32(1), 1)

        m_s[0] = jnp.int32(0)

        def refill_dumps():
            for j in range(BATCH_ROWS // 16):
                blist_v[pl.ds(j * 16, 16)] = BATCH + 16 * j + iota

        refill_dumps()

        losp = jnp.full((16,), lo, jnp.int32)
        hisp = jnp.full((16,), hi, jnp.int32)

        carry = jnp.int32(0)
        xbs = (xb0, xb1)
        xsems = (xsem0, xsem1)

        def xfire(piece, xb):
            pltpu.async_copy(
                x_hbm.at[pl.ds(piece * 2048, 2048)], xbs[xb], xsems[xb])

        xfire(0, 0)
        xfire(1, 1)
        for piece in range(8):
            xb = piece % 2
            pltpu.make_async_copy(
                x_hbm.at[pl.ds(0, 2048)], xbs[xb], xsems[xb]).wait()
            xb_v = xbs[xb]

            @pl.loop(0, 128, init_carry=carry)
            def match_loop(u, ptr):
                xv = xb_v[pl.ds(u * 16, 16)]
                bvec = piece * 2048 + u * 16 + iota
                m = jnp.logical_and(xv >= losp, xv < hisp)
                cs = plsc.cumsum(jnp.where(m, 1, 0).astype(jnp.int32))
                pos = cs + (ptr - 1)
                plsc.store_scatter(il_v, [pos], xv, mask=m)
                plsc.store_scatter(bl_v, [pos], bvec, mask=m)
                return ptr + cs[15]

            if piece + 2 < 8:
                xfire(piece + 2, xb)
            carry = match_loop
        nmatch = carry

        for j in range(4):
            hist_v[pl.ds(j * 16, 16)] = jnp.zeros((16,), jnp.int32)

        nmsp = jnp.full((16,), nmatch, jnp.int32)

        ngm = lax.shift_right_logical(nmatch + 15, 4)

        @pl.loop(0, ngm)
        def hist_loop(u):
            iv = il_v[pl.ds(u * 16, 16)]
            valid = (u * 16 + iota) < nmsp
            cvec = lax.shift_right_arithmetic(iv - losp, 9)
            plsc.addupdate_scatter(hist_v, [cvec], ones, mask=valid)

        acc = jnp.int32(0)
        for j in range(4):
            h = hist_v[pl.ds(j * 16, 16)]
            ha = lax.shift_left(lax.shift_right_logical(h + 15, 4), 4)
            csa = plsc.cumsum(ha)
            starts = csa - ha + acc
            for lane in range(16):
                base_s[j * 16 + lane] = starts[lane]
                fill_s[j * 16 + lane] = starts[lane]
                cnt_s[j * 16 + lane] = h[lane]
            acc = acc + csa[15]

        @pl.loop(0, ngm)
        def place_loop(u):
            iv = il_v[pl.ds(u * 16, 16)]
            bv = bl_v[pl.ds(u * 16, 16)]
            for lane in range(16):
                jj = u * 16 + lane

                @pl.when(jj < nmatch)
                def _():
                    i = iv[lane]
                    b = bv[lane]
                    cc = lax.shift_right_logical(i - lo, 9)
                    pos = fill_s[cc]
                    fill_s[cc] = pos + 1
                    psp = jnp.full((16,), pos, jnp.int32)
                    plsc.store_scatter(si_v, [psp], jnp.full((16,), i, jnp.int32), mask=lane0)
                    plsc.store_scatter(sb_v, [psp], jnp.full((16,), b, jnp.int32), mask=lane0)

        def flush():
            pltpu.async_copy(rows_v, out_hbm.at[blist_v], fsem).wait()
            refill_dumps()
            m_s[0] = jnp.int32(0)

        def emit(i, b, src_ref, col):
            mrow = m_s[0]
            csp = jnp.full((16,), col, jnp.int32)
            for q in range(4):
                v = plsc.load_gather(src_ref, [dvecs[q], csp])
                rows_v[mrow, pl.ds(q * 16, 16)] = v
            plsc.store_scatter(
                blist_v, [jnp.full((16,), mrow, jnp.int32)],
                jnp.full((16,), b, jnp.int32), mask=lane0)
            m_s[0] = mrow + 1

            @pl.when(mrow + 1 == BATCH_ROWS)
            def _():
                flush()

        NCHU = NCH0 + 1

        @pl.loop(0, (NCHU + 2) // 3)
        def sweep_loop(g):
            for bb in range(3):
                cidx = g * 3 + bb

                @pl.when(cidx < NCHU)
                def _():
                    pltpu.make_async_copy(
                        w_hbm.at[:, pl.ds(0, CW)], cbs[bb], sems[bb]
                    ).wait()
                    fire(cidx + 2, (bb + 2) % 3)
                    chunk_lo = lo + cidx * CW
                    cbase = base_s[cidx]
                    cn = cnt_s[cidx]
                    ng = lax.shift_right_logical(cn + 15, 4)

                    @pl.loop(0, ng)
                    def grp(u):
                        iv = si_v[pl.ds(cbase + u * 16, 16)]
                        bv = sb_v[pl.ds(cbase + u * 16, 16)]
                        for lane in range(16):
                            jj = u * 16 + lane

                            @pl.when(jj < cn)
                            def _():
                                emit(iv[lane], bv[lane], cbs[bb],
                                     iv[lane] - chunk_lo)

        for bb in (2, 0):
            pltpu.make_async_copy(
                w_hbm.at[:, pl.ds(0, CW)], cbs[bb], sems[bb]
            ).wait()

        tail_cp.wait()
        tbase = base_s[62]
        tn = cnt_s[62]
        ngt = lax.shift_right_logical(tn + 15, 4)

        @pl.loop(0, ngt)
        def tail_grp(u):
            iv = si_v[pl.ds(tbase + u * 16, 16)]
            bv = sb_v[pl.ds(tbase + u * 16, 16)]
            for lane in range(16):
                jj = u * 16 + lane

                @pl.when(jj < tn)
                def _():
                    emit(iv[lane], bv[lane], tail_v, iv[lane] - TAIL_START)

        @pl.when(m_s[0] > 0)
        def _():
            flush()

    out = body(x.astype(jnp.int32), wt, tail)
    return lax.slice(out, (0, 0), (BATCH, DIM))
